# initial kernel scaffold (unmeasured)
import sys

import jax
import jax.numpy as jnp
from jax import lax
from jax.experimental import pallas as pl
from jax.experimental.pallas import tpu as pltpu

N_DEV = 16

try:
    _ds = jax.devices()
    if _ds and _ds[0].platform == "tpu":
        import distributed_mesh_v7x as _dm

        _mesh = _dm.get_mesh("i", world_size=N_DEV)
        _ring = list(_mesh.devices.flat)
        _lines = [f"[kprobe] n_devices={len(_ds)} kind={_ds[0].device_kind}"]
        for _i, _d in enumerate(_ring):
            _nx = _ring[(_i + 1) % len(_ring)]
            _hop = sum(abs(a - b) for a, b in zip(_d.coords, _nx.coords))
            _lines.append(
                f"[kprobe] pos {_i}: {_d.coords} core={_d.core_on_chip}"
                f" -> {_nx.coords} manhattan={_hop}"
            )
        print("\n".join(_lines), file=sys.stderr)
except Exception as _e:
    print(f"[kprobe] failed: {_e!r}", file=sys.stderr)


def kernel(x, w_mat):
    m_per, k = x.shape
    _, n_per = w_mat.shape

    def body(x_ref, w_ref, out_ref, xg_ref, wb_ref, send_sems, recv_sems):
        my = lax.axis_index("i")
        left = lax.rem(my + N_DEV - 1, N_DEV)
        right = lax.rem(my + 1, N_DEV)

        barrier_sem = pltpu.get_barrier_semaphore()
        for nbr in (left, right):
            pl.semaphore_signal(
                barrier_sem,
                inc=1,
                device_id=(nbr,),
                device_id_type=pl.DeviceIdType.MESH,
            )
        pl.semaphore_wait(barrier_sem, 2)

        wb_ref[...] = w_ref[...].astype(jnp.bfloat16)
        xg_ref[0] = x_ref[...].astype(jnp.bfloat16)

        def gemm_slot(s):
            origin = lax.rem(my + N_DEV - s, N_DEV)
            y = jnp.dot(
                xg_ref[s], wb_ref[...], preferred_element_type=jnp.float32
            )
            out_ref[pl.ds(origin * m_per, m_per), :] = y * jax.nn.sigmoid(y)

        gemm_slot(0)

        for h in range(N_DEV - 1):
            rdma = pltpu.make_async_remote_copy(
                src_ref=xg_ref.at[h],
                dst_ref=xg_ref.at[h + 1],
                send_sem=send_sems.at[h],
                recv_sem=recv_sems.at[h],
                device_id=(right,),
                device_id_type=pl.DeviceIdType.MESH,
            )
            rdma.start()
            rdma.wait()
            gemm_slot(h + 1)

    return pl.pallas_call(
        body,
        out_shape=jax.ShapeDtypeStruct((N_DEV * m_per, n_per), jnp.float32),
        in_specs=[
            pl.BlockSpec(memory_space=pltpu.VMEM),
            pl.BlockSpec(memory_space=pltpu.VMEM),
        ],
        out_specs=pl.BlockSpec(memory_space=pltpu.VMEM),
        scratch_shapes=[
            pltpu.VMEM((N_DEV, m_per, k), jnp.bfloat16),
            pltpu.VMEM((k, n_per), jnp.bfloat16),
            pltpu.SemaphoreType.DMA((N_DEV - 1,)),
            pltpu.SemaphoreType.DMA((N_DEV - 1,)),
        ],
        compiler_params=pltpu.CompilerParams(collective_id=0),
    )(x, w_mat)


# baseline (device time: 388093 ns/iter reference)
import sys

import jax
import jax.numpy as jnp
from jax import lax
from jax.experimental import pallas as pl
from jax.experimental.pallas import tpu as pltpu

N_DEV = 16

try:
    _ds = jax.devices()
    if _ds and _ds[0].platform == "tpu":
        import distributed_mesh_v7x as _dm

        _mesh = _dm.get_mesh("i", world_size=N_DEV)
        _ring = list(_mesh.devices.flat)
        _lines = [f"[kprobe] n_devices={len(_ds)} kind={_ds[0].device_kind}"]
        for _i, _d in enumerate(_ring):
            _nx = _ring[(_i + 1) % len(_ring)]
            _hop = sum(abs(a - b) for a, b in zip(_d.coords, _nx.coords))
            _lines.append(
                f"[kprobe] pos {_i}: {_d.coords} core={_d.core_on_chip}"
                f" -> {_nx.coords} manhattan={_hop}"
            )
        print("\n".join(_lines), file=sys.stderr)
except Exception as _e:
    print(f"[kprobe] failed: {_e!r}", file=sys.stderr)


def kernel(x, w_mat):
    m_per, k = x.shape
    _, n_per = w_mat.shape

    def body(x_ref, w_ref, out_ref, xg_ref, wb_ref, send_sems, recv_sems):
        my = lax.axis_index("i")
        left = lax.rem(my + N_DEV - 1, N_DEV)
        right = lax.rem(my + 1, N_DEV)

        barrier_sem = pltpu.get_barrier_semaphore()
        for nbr in (left, right):
            pl.semaphore_signal(
                barrier_sem,
                inc=1,
                device_id=(nbr,),
                device_id_type=pl.DeviceIdType.MESH,
            )
        pl.semaphore_wait(barrier_sem, 2)

        wb_ref[...] = w_ref[...].astype(jnp.bfloat16)
        xg_ref[0] = x_ref[...].astype(jnp.bfloat16)

        def gemm_slot(s):
            origin = lax.rem(my + N_DEV - s, N_DEV)
            y = jnp.dot(
                xg_ref[s], wb_ref[...], preferred_element_type=jnp.float32
            )
            out_ref[pl.ds(origin * m_per, m_per), :] = y * jax.nn.sigmoid(y)

        gemm_slot(0)

        for h in range(N_DEV - 1):
            rdma = pltpu.make_async_remote_copy(
                src_ref=xg_ref.at[h],
                dst_ref=xg_ref.at[h + 1],
                send_sem=send_sems.at[h],
                recv_sem=recv_sems.at[h],
                device_id=(right,),
                device_id_type=pl.DeviceIdType.MESH,
            )
            rdma.start()
            rdma.wait()
            gemm_slot(h + 1)

    return pl.pallas_call(
        body,
        out_shape=jax.ShapeDtypeStruct((N_DEV * m_per, n_per), jnp.float32),
        in_specs=[
            pl.BlockSpec(memory_space=pltpu.VMEM),
            pl.BlockSpec(memory_space=pltpu.VMEM),
        ],
        out_specs=pl.BlockSpec(memory_space=pltpu.VMEM),
        scratch_shapes=[
            pltpu.VMEM((N_DEV, m_per, k), jnp.bfloat16),
            pltpu.VMEM((k, n_per), jnp.bfloat16),
            pltpu.SemaphoreType.DMA((N_DEV - 1,)),
            pltpu.SemaphoreType.DMA((N_DEV - 1,)),
        ],
        compiler_params=pltpu.CompilerParams(
            collective_id=0, vmem_limit_bytes=60 * 1024 * 1024
        ),
    )(x, w_mat)


# device time: 207044 ns/iter; 1.8744x vs baseline; 1.8744x over previous
import sys

import jax
import jax.numpy as jnp
from jax import lax
from jax.experimental import pallas as pl
from jax.experimental.pallas import tpu as pltpu

N_DEV = 16

try:
    _ds = jax.devices()
    if _ds and _ds[0].platform == "tpu":
        import distributed_mesh_v7x as _dm

        _mesh = _dm.get_mesh("i", world_size=N_DEV)
        _ring = list(_mesh.devices.flat)
        _lines = [f"[kprobe] n_devices={len(_ds)} kind={_ds[0].device_kind}"]
        for _i, _d in enumerate(_ring):
            _nx = _ring[(_i + 1) % len(_ring)]
            _hop = sum(abs(a - b) for a, b in zip(_d.coords, _nx.coords))
            _lines.append(
                f"[kprobe] pos {_i}: {_d.coords} core={_d.core_on_chip}"
                f" -> {_nx.coords} manhattan={_hop}"
            )
        print("\n".join(_lines), file=sys.stderr)
except Exception as _e:
    print(f"[kprobe] failed: {_e!r}", file=sys.stderr)


def kernel(x, w_mat):
    m_per, k = x.shape
    _, n_per = w_mat.shape

    def body(x_ref, w_ref, out_ref, xg_ref, wb_ref, r_send, r_recv, l_send, l_recv):
        my = lax.axis_index("i")
        left = lax.rem(my + N_DEV - 1, N_DEV)
        right = lax.rem(my + 1, N_DEV)

        barrier_sem = pltpu.get_barrier_semaphore()
        for nbr in (left, right):
            pl.semaphore_signal(
                barrier_sem,
                inc=1,
                device_id=(nbr,),
                device_id_type=pl.DeviceIdType.MESH,
            )
        pl.semaphore_wait(barrier_sem, 2)

        wb_ref[...] = w_ref[...].astype(jnp.bfloat16)
        xg_ref[0] = x_ref[...].astype(jnp.bfloat16)

        def gemm_slot(s):
            origin = lax.rem(my + N_DEV - s, N_DEV)
            y = jnp.dot(
                xg_ref[s], wb_ref[...], preferred_element_type=jnp.float32
            )
            out_ref[pl.ds(origin * m_per, m_per), :] = y * jax.nn.sigmoid(y)

        R_HOPS = N_DEV // 2
        L_HOPS = N_DEV - 1 - R_HOPS

        r = [
            pltpu.make_async_remote_copy(
                src_ref=xg_ref.at[h],
                dst_ref=xg_ref.at[h + 1],
                send_sem=r_send.at[h],
                recv_sem=r_recv.at[h],
                device_id=(right,),
                device_id_type=pl.DeviceIdType.MESH,
            )
            for h in range(R_HOPS)
        ]
        l = [
            pltpu.make_async_remote_copy(
                src_ref=xg_ref.at[(N_DEV - h) % N_DEV],
                dst_ref=xg_ref.at[N_DEV - 1 - h],
                send_sem=l_send.at[h],
                recv_sem=l_recv.at[h],
                device_id=(left,),
                device_id_type=pl.DeviceIdType.MESH,
            )
            for h in range(L_HOPS)
        ]

        r[0].start()
        l[0].start()
        gemm_slot(0)

        for h in range(R_HOPS):
            r[h].wait_recv()
            if h + 1 < R_HOPS:
                r[h + 1].start()
            if h < L_HOPS:
                l[h].wait_recv()
                if h + 1 < L_HOPS:
                    l[h + 1].start()
            gemm_slot(h + 1)
            if h < L_HOPS:
                gemm_slot(N_DEV - 1 - h)

        for h in range(R_HOPS):
            r[h].wait_send()
        for h in range(L_HOPS):
            l[h].wait_send()

    return pl.pallas_call(
        body,
        out_shape=jax.ShapeDtypeStruct((N_DEV * m_per, n_per), jnp.float32),
        in_specs=[
            pl.BlockSpec(memory_space=pltpu.VMEM),
            pl.BlockSpec(memory_space=pltpu.VMEM),
        ],
        out_specs=pl.BlockSpec(memory_space=pltpu.VMEM),
        scratch_shapes=[
            pltpu.VMEM((N_DEV, m_per, k), jnp.bfloat16),
            pltpu.VMEM((k, n_per), jnp.bfloat16),
            pltpu.SemaphoreType.DMA((N_DEV // 2,)),
            pltpu.SemaphoreType.DMA((N_DEV // 2,)),
            pltpu.SemaphoreType.DMA((N_DEV // 2 - 1,)),
            pltpu.SemaphoreType.DMA((N_DEV // 2 - 1,)),
        ],
        compiler_params=pltpu.CompilerParams(
            collective_id=0, vmem_limit_bytes=60 * 1024 * 1024
        ),
    )(x, w_mat)


# device time: 186175 ns/iter; 2.0846x vs baseline; 1.1121x over previous
import sys

import jax
import jax.numpy as jnp
from jax import lax
from jax.experimental import pallas as pl
from jax.experimental.pallas import tpu as pltpu

N_DEV = 16

try:
    _ds = jax.devices()
    if _ds and _ds[0].platform == "tpu":
        import distributed_mesh_v7x as _dm

        _mesh = _dm.get_mesh("i", world_size=N_DEV)
        _ring = list(_mesh.devices.flat)
        _lines = [f"[kprobe] n_devices={len(_ds)} kind={_ds[0].device_kind}"]
        for _i, _d in enumerate(_ring):
            _nx = _ring[(_i + 1) % len(_ring)]
            _hop = sum(abs(a - b) for a, b in zip(_d.coords, _nx.coords))
            _lines.append(
                f"[kprobe] pos {_i}: {_d.coords} core={_d.core_on_chip}"
                f" -> {_nx.coords} manhattan={_hop}"
            )
        print("\n".join(_lines), file=sys.stderr)
except Exception as _e:
    print(f"[kprobe] failed: {_e!r}", file=sys.stderr)


def kernel(x, w_mat):
    m_per, k = x.shape
    _, n_per = w_mat.shape

    def body(x_ref, w_ref, out_ref, xg_ref, wb_ref, r_send, r_recv, l_send, l_recv):
        my = lax.axis_index("i")
        left = lax.rem(my + N_DEV - 1, N_DEV)
        right = lax.rem(my + 1, N_DEV)

        barrier_sem = pltpu.get_barrier_semaphore()
        for nbr in (left, right):
            pl.semaphore_signal(
                barrier_sem,
                inc=1,
                device_id=(nbr,),
                device_id_type=pl.DeviceIdType.MESH,
            )
        pl.semaphore_wait(barrier_sem, 2)

        half = m_per // 2

        xg_ref[0, : half] = x_ref[: half, :].astype(jnp.bfloat16)
        xg_ref[0, half :] = x_ref[half :, :].astype(jnp.bfloat16)

        def gemm_slot(s):
            origin = lax.rem(my + N_DEV - s, N_DEV)
            y = jnp.dot(
                xg_ref[s], wb_ref[...], preferred_element_type=jnp.float32
            )
            out_ref[pl.ds(origin * m_per, m_per), :] = y * jax.nn.sigmoid(y)

        HOPS = N_DEV // 2

        def piece(s, j):
            return xg_ref.at[s, pl.ds(j * half, half)]

        def msgs(nbr, src_slot, dst_slot, send_s, recv_s, h, keep_j):
            return [
                pltpu.make_async_remote_copy(
                    src_ref=piece(src_slot, j),
                    dst_ref=piece(dst_slot, j),
                    send_sem=send_s.at[2 * h + j],
                    recv_sem=recv_s.at[2 * h + j],
                    device_id=(nbr,),
                    device_id_type=pl.DeviceIdType.MESH,
                )
                if (h < HOPS - 1 or j == keep_j)
                else None
                for j in range(2)
            ]

        r = [
            msgs(right, h, h + 1, r_send, r_recv, h, keep_j=0)
            for h in range(HOPS)
        ]
        l = [
            msgs(left, (N_DEV - h) % N_DEV, N_DEV - 1 - h, l_send, l_recv,
                 h, keep_j=1)
            for h in range(HOPS)
        ]

        for j in range(2):
            if r[0][j] is not None:
                r[0][j].start()
            if l[0][j] is not None:
                l[0][j].start()
        wb_ref[...] = w_ref[...].astype(jnp.bfloat16)
        gemm_slot(0)

        for h in range(HOPS):
            for j in range(2):
                if r[h][j] is not None:
                    r[h][j].wait_recv()
                    if h + 1 < HOPS and r[h + 1][j] is not None:
                        r[h + 1][j].start()
                if l[h][j] is not None:
                    l[h][j].wait_recv()
                    if h + 1 < HOPS and l[h + 1][j] is not None:
                        l[h + 1][j].start()
            if h < HOPS - 1:
                gemm_slot(h + 1)
                gemm_slot(N_DEV - 1 - h)
        gemm_slot(HOPS)

        for h in range(HOPS):
            for j in range(2):
                if r[h][j] is not None:
                    r[h][j].wait_send()
                if l[h][j] is not None:
                    l[h][j].wait_send()

    return pl.pallas_call(
        body,
        out_shape=jax.ShapeDtypeStruct((N_DEV * m_per, n_per), jnp.float32),
        in_specs=[
            pl.BlockSpec(memory_space=pltpu.VMEM),
            pl.BlockSpec(memory_space=pltpu.VMEM),
        ],
        out_specs=pl.BlockSpec(memory_space=pltpu.VMEM),
        scratch_shapes=[
            pltpu.VMEM((N_DEV, m_per, k), jnp.bfloat16),
            pltpu.VMEM((k, n_per), jnp.bfloat16),
            pltpu.SemaphoreType.DMA((N_DEV,)),
            pltpu.SemaphoreType.DMA((N_DEV,)),
            pltpu.SemaphoreType.DMA((N_DEV,)),
            pltpu.SemaphoreType.DMA((N_DEV,)),
        ],
        compiler_params=pltpu.CompilerParams(
            collective_id=0, vmem_limit_bytes=60 * 1024 * 1024
        ),
    )(x, w_mat)


# device time: 185962 ns/iter; 2.0869x vs baseline; 1.0011x over previous
import sys

import jax
import jax.numpy as jnp
from jax import lax
from jax.experimental import pallas as pl
from jax.experimental.pallas import tpu as pltpu

N_DEV = 16

try:
    _ds = jax.devices()
    if _ds and _ds[0].platform == "tpu":
        import distributed_mesh_v7x as _dm

        _mesh = _dm.get_mesh("i", world_size=N_DEV)
        _ring = list(_mesh.devices.flat)
        _lines = [f"[kprobe] n_devices={len(_ds)} kind={_ds[0].device_kind}"]
        for _i, _d in enumerate(_ring):
            _nx = _ring[(_i + 1) % len(_ring)]
            _hop = sum(abs(a - b) for a, b in zip(_d.coords, _nx.coords))
            _lines.append(
                f"[kprobe] pos {_i}: {_d.coords} core={_d.core_on_chip}"
                f" -> {_nx.coords} manhattan={_hop}"
            )
        print("\n".join(_lines), file=sys.stderr)
except Exception as _e:
    print(f"[kprobe] failed: {_e!r}", file=sys.stderr)


def kernel(x, w_mat):
    m_per, k = x.shape
    _, n_per = w_mat.shape

    def body(x_ref, w_ref, out_ref, xg_ref, wb_ref, r_send, r_recv, l_send, l_recv):
        my = lax.axis_index("i")
        left = lax.rem(my + N_DEV - 1, N_DEV)
        right = lax.rem(my + 1, N_DEV)

        barrier_sem = pltpu.get_barrier_semaphore()
        for nbr in (left, right):
            pl.semaphore_signal(
                barrier_sem,
                inc=1,
                device_id=(nbr,),
                device_id_type=pl.DeviceIdType.MESH,
            )

        half = m_per // 2

        xg_ref[0, : half] = x_ref[: half, :].astype(jnp.bfloat16)
        xg_ref[0, half :] = x_ref[half :, :].astype(jnp.bfloat16)

        def gemm_slot(s):
            origin = lax.rem(my + N_DEV - s, N_DEV)
            y = jnp.dot(
                xg_ref[s], wb_ref[...], preferred_element_type=jnp.float32
            )
            out_ref[pl.ds(origin * m_per, m_per), :] = y * jax.nn.sigmoid(y)

        HOPS = N_DEV // 2

        def piece(s, j):
            return xg_ref.at[s, pl.ds(j * half, half)]

        def msgs(nbr, src_slot, dst_slot, send_s, recv_s, h, keep_j):
            return [
                pltpu.make_async_remote_copy(
                    src_ref=piece(src_slot, j),
                    dst_ref=piece(dst_slot, j),
                    send_sem=send_s.at[2 * h + j],
                    recv_sem=recv_s.at[2 * h + j],
                    device_id=(nbr,),
                    device_id_type=pl.DeviceIdType.MESH,
                )
                if (h < HOPS - 1 or j == keep_j)
                else None
                for j in range(2)
            ]

        r = [
            msgs(right, h, h + 1, r_send, r_recv, h, keep_j=0)
            for h in range(HOPS)
        ]
        l = [
            msgs(left, (N_DEV - h) % N_DEV, N_DEV - 1 - h, l_send, l_recv,
                 h, keep_j=1)
            for h in range(HOPS)
        ]

        wb_ref[...] = w_ref[...].astype(jnp.bfloat16)
        gemm_slot(0)

        pl.semaphore_wait(barrier_sem, 2)
        for j in range(2):
            if r[0][j] is not None:
                r[0][j].start()
            if l[0][j] is not None:
                l[0][j].start()

        for h in range(HOPS):
            for j in range(2):
                if r[h][j] is not None:
                    r[h][j].wait_recv()
                    if h + 1 < HOPS and r[h + 1][j] is not None:
                        r[h + 1][j].start()
                if l[h][j] is not None:
                    l[h][j].wait_recv()
                    if h + 1 < HOPS and l[h + 1][j] is not None:
                        l[h + 1][j].start()
            if h < HOPS - 1:
                gemm_slot(h + 1)
                gemm_slot(N_DEV - 1 - h)
        gemm_slot(HOPS)

        for h in range(HOPS):
            for j in range(2):
                if r[h][j] is not None:
                    r[h][j].wait_send()
                if l[h][j] is not None:
                    l[h][j].wait_send()

    return pl.pallas_call(
        body,
        out_shape=jax.ShapeDtypeStruct((N_DEV * m_per, n_per), jnp.float32),
        in_specs=[
            pl.BlockSpec(memory_space=pltpu.VMEM),
            pl.BlockSpec(memory_space=pltpu.VMEM),
        ],
        out_specs=pl.BlockSpec(memory_space=pltpu.VMEM),
        scratch_shapes=[
            pltpu.VMEM((N_DEV, m_per, k), jnp.bfloat16),
            pltpu.VMEM((k, n_per), jnp.bfloat16),
            pltpu.SemaphoreType.DMA((N_DEV,)),
            pltpu.SemaphoreType.DMA((N_DEV,)),
            pltpu.SemaphoreType.DMA((N_DEV,)),
            pltpu.SemaphoreType.DMA((N_DEV,)),
        ],
        compiler_params=pltpu.CompilerParams(
            collective_id=0, vmem_limit_bytes=60 * 1024 * 1024
        ),
    )(x, w_mat)


# device time: 183971 ns/iter; 2.1095x vs baseline; 1.0108x over previous
import sys

import jax
import jax.numpy as jnp
from jax import lax
from jax.experimental import pallas as pl
from jax.experimental.pallas import tpu as pltpu

N_DEV = 16

PERM = (0, 1, 5, 9, 13, 14, 10, 6, 2, 3, 7, 11, 15, 12, 8, 4)
INV = tuple(PERM.index(i) for i in range(N_DEV))

try:
    _ds = jax.devices()
    if _ds and _ds[0].platform == "tpu":
        import distributed_mesh_v7x as _dm

        _mesh = _dm.get_mesh("i", world_size=N_DEV)
        _ring = list(_mesh.devices.flat)
        _lines = [f"[kprobe] n_devices={len(_ds)} kind={_ds[0].device_kind}"]
        for _i, _d in enumerate(_ring):
            _nx = _ring[(_i + 1) % len(_ring)]
            _hop = sum(abs(a - b) for a, b in zip(_d.coords, _nx.coords))
            _lines.append(
                f"[kprobe] pos {_i}: {_d.coords} core={_d.core_on_chip}"
                f" -> {_nx.coords} manhattan={_hop}"
            )
        print("\n".join(_lines), file=sys.stderr)
except Exception as _e:
    print(f"[kprobe] failed: {_e!r}", file=sys.stderr)


def kernel(x, w_mat):
    m_per, k = x.shape
    _, n_per = w_mat.shape

    def body(x_ref, w_ref, out_ref, xg_ref, wb_ref, r_send, r_recv, l_send, l_recv):
        my = lax.axis_index("i")

        def lut(table, idx):
            acc = jnp.int32(table[0])
            for k in range(1, N_DEV):
                acc = lax.select(idx == k, jnp.int32(table[k]), acc)
            return acc

        rho = lut(INV, my)
        left = lut(PERM, lax.rem(rho + N_DEV - 1, N_DEV))
        right = lut(PERM, lax.rem(rho + 1, N_DEV))

        barrier_sem = pltpu.get_barrier_semaphore()
        for nbr in (left, right):
            pl.semaphore_signal(
                barrier_sem,
                inc=1,
                device_id=(nbr,),
                device_id_type=pl.DeviceIdType.MESH,
            )

        half = m_per // 2

        xg_ref[0, : half] = x_ref[: half, :].astype(jnp.bfloat16)
        xg_ref[0, half :] = x_ref[half :, :].astype(jnp.bfloat16)

        def gemm_slot(s):
            origin = lut(PERM, lax.rem(rho + N_DEV - s, N_DEV))
            y = jnp.dot(
                xg_ref[s], wb_ref[...], preferred_element_type=jnp.float32
            )
            out_ref[pl.ds(origin * m_per, m_per), :] = y * jax.nn.sigmoid(y)

        HOPS = N_DEV // 2

        def piece(s, j):
            return xg_ref.at[s, pl.ds(j * half, half)]

        def msgs(nbr, src_slot, dst_slot, send_s, recv_s, h, keep_j):
            return [
                pltpu.make_async_remote_copy(
                    src_ref=piece(src_slot, j),
                    dst_ref=piece(dst_slot, j),
                    send_sem=send_s.at[2 * h + j],
                    recv_sem=recv_s.at[2 * h + j],
                    device_id=(nbr,),
                    device_id_type=pl.DeviceIdType.MESH,
                )
                if (h < HOPS - 1 or j == keep_j)
                else None
                for j in range(2)
            ]

        r = [
            msgs(right, h, h + 1, r_send, r_recv, h, keep_j=0)
            for h in range(HOPS)
        ]
        l = [
            msgs(left, (N_DEV - h) % N_DEV, N_DEV - 1 - h, l_send, l_recv,
                 h, keep_j=1)
            for h in range(HOPS)
        ]

        wb_ref[...] = w_ref[...].astype(jnp.bfloat16)
        gemm_slot(0)

        pl.semaphore_wait(barrier_sem, 2)
        for j in range(2):
            if r[0][j] is not None:
                r[0][j].start()
            if l[0][j] is not None:
                l[0][j].start()

        for h in range(HOPS):
            for j in range(2):
                if r[h][j] is not None:
                    r[h][j].wait_recv()
                    if h + 1 < HOPS and r[h + 1][j] is not None:
                        r[h + 1][j].start()
                if l[h][j] is not None:
                    l[h][j].wait_recv()
                    if h + 1 < HOPS and l[h + 1][j] is not None:
                        l[h + 1][j].start()
            if h < HOPS - 1:
                gemm_slot(h + 1)
                gemm_slot(N_DEV - 1 - h)
        gemm_slot(HOPS)

        for h in range(HOPS):
            for j in range(2):
                if r[h][j] is not None:
                    r[h][j].wait_send()
                if l[h][j] is not None:
                    l[h][j].wait_send()

    return pl.pallas_call(
        body,
        out_shape=jax.ShapeDtypeStruct((N_DEV * m_per, n_per), jnp.float32),
        in_specs=[
            pl.BlockSpec(memory_space=pltpu.VMEM),
            pl.BlockSpec(memory_space=pltpu.VMEM),
        ],
        out_specs=pl.BlockSpec(memory_space=pltpu.VMEM),
        scratch_shapes=[
            pltpu.VMEM((N_DEV, m_per, k), jnp.bfloat16),
            pltpu.VMEM((k, n_per), jnp.bfloat16),
            pltpu.SemaphoreType.DMA((N_DEV,)),
            pltpu.SemaphoreType.DMA((N_DEV,)),
            pltpu.SemaphoreType.DMA((N_DEV,)),
            pltpu.SemaphoreType.DMA((N_DEV,)),
        ],
        compiler_params=pltpu.CompilerParams(
            collective_id=0, vmem_limit_bytes=60 * 1024 * 1024
        ),
    )(x, w_mat)
